# baseline (speedup 1.0000x reference)
import functools

import jax
import jax.numpy as jnp
from jax import lax
from jax.experimental import pallas as pl
from jax.experimental.pallas import tpu as pltpu
from jax.experimental.pallas import tpu_sc as plsc

_EPS = 1e-5
_NC, _NS = 2, 16
_NW = _NC * _NS
_CW = 128


def _sc_mesh():
    return plsc.VectorSubcoreMesh(core_axis_name="c", subcore_axis_name="s")


def _row_split(N):
    rows_a = (N // _NS) & ~7
    tail = N - _NS * rows_a
    return rows_a, tail


def _make_deg1_kernel(N, n_pw):
    rows_a, tail = _row_split(N)
    Np = N + 8
    n_half = n_pw // 2

    @functools.partial(
        pl.kernel,
        out_type=jax.ShapeDtypeStruct((_NC * N,), jnp.float32),
        mesh=_sc_mesh(),
        scratch_types=[
            pltpu.VMEM((n_half, _CW), jnp.int32),
            pltpu.VMEM((_CW,), jnp.float32),
            pltpu.VMEM((rows_a + tail + 8,), jnp.float32),
            pltpu.VMEM_SHARED((Np,), jnp.float32),
            pltpu.SemaphoreType.DMA,
            pltpu.SemaphoreType.DMA,
        ],
    )
    def deg_kernel(ones_hbm, dst_hbm, zeros_hbm, out_hbm,
                   dst_v, ones_v, stage_v, acc_sh, ss0, ss1):
        c = lax.axis_index("c")
        s = lax.axis_index("s")
        w = s * _NC + c
        t0 = s * rows_a
        pltpu.sync_copy(zeros_hbm.at[pl.ds(0, rows_a + tail + 8)], stage_v)
        pltpu.sync_copy(stage_v.at[pl.ds(0, rows_a)],
                        acc_sh.at[pl.ds(t0, rows_a)])

        @pl.when(s == _NS - 1)
        def _():
            pltpu.sync_copy(stage_v.at[pl.ds(rows_a, tail + 8)],
                            acc_sh.at[pl.ds(_NS * rows_a, tail + 8)])

        pltpu.sync_copy(ones_hbm, ones_v)
        plsc.subcore_barrier()
        sems = (ss0, ss1)

        def issue(i, b):
            pltpu.async_copy(ones_v, acc_sh.at[dst_v.at[i]], sems[b], add=True)

        def wait(i, b):
            pltpu.make_async_copy(ones_v, acc_sh.at[dst_v.at[i]],
                                  sems[b]).wait()

        def run_half(half):
            row_base = w * n_pw + half * n_half
            pltpu.sync_copy(dst_hbm.at[pl.ds(row_base, n_half)], dst_v)
            issue(0, 0)
            issue(1, 1)

            def body(p, carry):
                for q in range(2):
                    i = 2 * p + q + 2
                    wait(i - 2, q)
                    issue(i, q)
                return carry

            lax.fori_loop(0, (n_half - 2) // 2, body, 0)
            for i in range(2 + 2 * ((n_half - 2) // 2), n_half):
                wait(i - 2, i % 2)
                issue(i, i % 2)
            for i in range(n_half - 2, n_half):
                wait(i, i % 2)

        run_half(0)
        run_half(1)
        plsc.subcore_barrier()
        pltpu.sync_copy(acc_sh.at[pl.ds(t0, rows_a)],
                        stage_v.at[pl.ds(0, rows_a)])
        pltpu.sync_copy(stage_v.at[pl.ds(0, rows_a)],
                        out_hbm.at[pl.ds(c * N + t0, rows_a)])

        @pl.when(s == _NS - 1)
        def _():
            pltpu.sync_copy(acc_sh.at[pl.ds(_NS * rows_a, tail)],
                            stage_v.at[pl.ds(0, tail)])
            pltpu.sync_copy(stage_v.at[pl.ds(0, tail)],
                            out_hbm.at[pl.ds(c * N + _NS * rows_a, tail)])

    return deg_kernel


def _make_scatter_kernel(N, n_pw, H, with_gather=True):
    rows_a, tail = _row_split(N)
    Np = N + 8
    n_half = n_pw // 2

    scratch = [
        pltpu.VMEM((n_half, _CW), jnp.int32),
        pltpu.VMEM((_CW, H), jnp.float32),
        pltpu.VMEM((_CW, H), jnp.float32),
        pltpu.VMEM_SHARED((Np, H), jnp.float32),
        pltpu.SemaphoreType.DMA,
        pltpu.SemaphoreType.DMA,
        pltpu.SemaphoreType.DMA,
        pltpu.SemaphoreType.DMA,
    ]
    if with_gather:
        scratch.insert(0, pltpu.VMEM((n_half, _CW), jnp.int32))

    @functools.partial(
        pl.kernel,
        out_type=jax.ShapeDtypeStruct((_NC, N, H), jnp.float32),
        mesh=_sc_mesh(),
        scratch_types=scratch,
    )
    def scatter_kernel(*refs):
        if with_gather:
            (hp_hbm, src_hbm, dst_hbm, zeros_hbm, out_hbm,
             src_v, dst_v, r0b, r1b, acc_sh, sg0, sg1, ss0, ss1) = refs
        else:
            (ones_hbm, dst_hbm, zeros_hbm, out_hbm,
             dst_v, r0b, r1b, acc_sh, sg0, sg1, ss0, ss1) = refs

        c = lax.axis_index("c")
        s = lax.axis_index("s")
        w = s * _NC + c
        t0 = s * rows_a
        pltpu.sync_copy(zeros_hbm.at[pl.ds(t0, rows_a)],
                        acc_sh.at[pl.ds(t0, rows_a)])

        @pl.when(s == _NS - 1)
        def _():
            pltpu.sync_copy(zeros_hbm.at[pl.ds(_NS * rows_a, tail)],
                            acc_sh.at[pl.ds(_NS * rows_a, tail)])

        if not with_gather:
            pltpu.sync_copy(ones_hbm, r0b)
        plsc.subcore_barrier()

        gbufs = ((r0b, sg0, ss0), (r1b, sg1, ss1))

        def run_half(half):
            row_base = w * n_pw + half * n_half
            pltpu.sync_copy(dst_hbm.at[pl.ds(row_base, n_half)], dst_v)
            if with_gather:
                pltpu.sync_copy(src_hbm.at[pl.ds(row_base, n_half)], src_v)

                def start_gather(i, b):
                    rows_v, sg, _ = gbufs[b]
                    pltpu.async_copy(hp_hbm.at[src_v.at[i]], rows_v, sg)

                def wait_gather(i, b):
                    rows_v, sg, _ = gbufs[b]
                    pltpu.make_async_copy(hp_hbm.at[src_v.at[i]], rows_v,
                                          sg).wait()

                def issue_scatter(i, b):
                    rows_v, _, ss = gbufs[b]
                    pltpu.async_copy(rows_v, acc_sh.at[dst_v.at[i]], ss,
                                     add=True)

                def wait_scatter(i, b):
                    rows_v, _, ss = gbufs[b]
                    pltpu.make_async_copy(rows_v, acc_sh.at[dst_v.at[i]],
                                          ss).wait()

                start_gather(0, 0)
                wait_gather(0, 0)
                issue_scatter(0, 0)
                start_gather(1, 1)

                def mid(i, b):
                    bp = b ^ 1
                    wait_gather(i, b)
                    issue_scatter(i, b)
                    wait_scatter(i - 1, bp)

                    @pl.when(i + 1 < n_half)
                    def _():
                        start_gather(i + 1, bp)

                def body(p, carry):
                    mid(2 * p + 1, 1)
                    mid(2 * p + 2, 0)
                    return carry

                lax.fori_loop(0, (n_half - 1) // 2, body, 0)
                for i in range(1 + 2 * ((n_half - 1) // 2), n_half):
                    mid(i, i % 2)
                wait_scatter(n_half - 1, (n_half - 1) % 2)
            else:
                def issue_scatter(i, b):
                    _, _, ss = gbufs[b]
                    pltpu.async_copy(r0b, acc_sh.at[dst_v.at[i]], ss, add=True)

                def wait_scatter(i, b):
                    _, _, ss = gbufs[b]
                    pltpu.make_async_copy(r0b, acc_sh.at[dst_v.at[i]],
                                          ss).wait()

                issue_scatter(0, 0)
                issue_scatter(1, 1)

                def body(p, carry):
                    for q in range(2):
                        i = 2 * p + q + 2
                        wait_scatter(i - 2, q)
                        issue_scatter(i, q)
                    return carry

                lax.fori_loop(0, (n_half - 2) // 2, body, 0)
                for i in range(2 + 2 * ((n_half - 2) // 2), n_half):
                    wait_scatter(i - 2, i % 2)
                    issue_scatter(i, i % 2)
                for i in range(n_half - 2, n_half):
                    wait_scatter(i, i % 2)

        run_half(0)
        run_half(1)

        plsc.subcore_barrier()
        pltpu.sync_copy(acc_sh.at[pl.ds(t0, rows_a)],
                        out_hbm.at[c, pl.ds(t0, rows_a)])

        @pl.when(s == _NS - 1)
        def _():
            pltpu.sync_copy(acc_sh.at[pl.ds(_NS * rows_a, tail)],
                            out_hbm.at[c, pl.ds(_NS * rows_a, tail)])

    return scatter_kernel


def _tc0_body(x_ref, w_ref, d0_ref, d1_ref, hp_ref, dinv_ref):
    N = x_ref.shape[0]
    deg = d0_ref[:, 0:1] + d1_ref[:, 0:1] + 1.0
    dinv = lax.rsqrt(deg)
    dinv_ref[...] = dinv
    pre = jnp.dot(x_ref[...], w_ref[...], preferred_element_type=jnp.float32)
    hp_ref[pl.ds(0, N), :] = pre * dinv
    hp_ref[pl.ds(N, 8), :] = jnp.zeros((8, hp_ref.shape[1]), jnp.float32)


def _tc_mid_body(s0_ref, s1_ref, hp_ref, dinv_ref, b_ref, g_ref, be_ref,
                 w_ref, hn_ref, hnp_ref):
    N = s0_ref.shape[0]
    dinv = dinv_ref[...]
    z = dinv * (s0_ref[...] + s1_ref[...] + hp_ref[pl.ds(0, N), :]) + b_ref[...]
    mu = jnp.mean(z, axis=0, keepdims=True)
    var = jnp.mean((z - mu) ** 2, axis=0, keepdims=True)
    hn = jnp.maximum((z - mu) * lax.rsqrt(var + _EPS) * g_ref[...] + be_ref[...],
                     0.0)
    hn_ref[...] = hn
    hnp_ref[pl.ds(0, N), :] = jnp.dot(hn, w_ref[...],
                                      preferred_element_type=jnp.float32) * dinv
    hnp_ref[pl.ds(N, 8), :] = jnp.zeros((8, hnp_ref.shape[1]), jnp.float32)


def _tc_final_body(s0_ref, s1_ref, hp_ref, dinv_ref, b_ref, g_ref, be_ref,
                   res_ref, wfc_ref, bfc_ref, out_ref):
    N = s0_ref.shape[0]
    dinv = dinv_ref[...]
    z = dinv * (s0_ref[...] + s1_ref[...] + hp_ref[pl.ds(0, N), :]) + b_ref[...]
    mu = jnp.mean(z, axis=0, keepdims=True)
    var = jnp.mean((z - mu) ** 2, axis=0, keepdims=True)
    h = jnp.maximum((z - mu) * lax.rsqrt(var + _EPS) * g_ref[...] + be_ref[...],
                    0.0)
    h = h + res_ref[...]
    logits = jnp.dot(h, wfc_ref[...], preferred_element_type=jnp.float32)
    out_ref[...] = jax.nn.sigmoid(logits + bfc_ref[...]) * 10.0


def _tc_call(body, out_shapes, *args):
    return pl.pallas_call(body, out_shape=out_shapes)(*args)


def kernel(x, edge_index, W1, b1, g1, be1, W2, b2, g2, be2, W3, b3, g3, be3,
           Wfc, bfc):
    N, D = x.shape
    H = W1.shape[1]
    E = edge_index.shape[1]

    grain = _NW * _CW * 8
    E_pad = -(-E // grain) * grain
    pad = E_pad - E
    pad_src = N + (jnp.arange(pad, dtype=jnp.int32) % 8)
    pad_dst = (jnp.arange(pad, dtype=jnp.int32) * 997) % N
    src_m = jnp.concatenate([edge_index[0], pad_src]).reshape(E_pad // _CW, _CW)
    dst_m = jnp.concatenate([edge_index[1], pad_dst]).reshape(E_pad // _CW, _CW)
    dst_deg = jnp.concatenate(
        [edge_index[1], pad_src]).reshape(E_pad // _CW, _CW)
    n_pw = E_pad // _CW // _NW

    n_real = E // _CW
    n_padr = E_pad // _CW - n_real
    pi = []
    r, p = 0, n_real
    for wk in range(_NW):
        pc = n_padr // _NW + (1 if wk < n_padr % _NW else 0)
        rc = n_pw - pc
        pi.extend(range(r, r + rc)); r += rc
        pi.extend(range(p, p + pc)); p += pc
    pi = jnp.asarray(pi, jnp.int32)
    src_m = src_m[pi]
    dst_m = dst_m[pi]
    dst_deg = dst_deg[pi]

    zerosNH = jnp.zeros((N, H), jnp.float32)
    zeros1 = jnp.zeros((N + 8,), jnp.float32)
    ones1 = jnp.ones((_CW,), jnp.float32)

    b1r, g1r, be1r = b1.reshape(1, H), g1.reshape(1, H), be1.reshape(1, H)
    b2r, g2r, be2r = b2.reshape(1, H), g2.reshape(1, H), be2.reshape(1, H)
    b3r, g3r, be3r = b3.reshape(1, H), g3.reshape(1, H), be3.reshape(1, H)
    bfcr = bfc.reshape(1, 1)

    deg_k = _make_deg1_kernel(N, n_pw)
    scat_k = _make_scatter_kernel(N, n_pw, H, with_gather=True)

    degp = deg_k(ones1, dst_deg, zeros1).reshape(_NC, N, 1)

    h1p, dinv = _tc_call(
        _tc0_body,
        (jax.ShapeDtypeStruct((N + 8, H), jnp.float32),
         jax.ShapeDtypeStruct((N, 1), jnp.float32)),
        x, W1, degp[0], degp[1])

    S1 = scat_k(h1p, src_m, dst_m, zerosNH)
    h1, h2p = _tc_call(
        _tc_mid_body,
        (jax.ShapeDtypeStruct((N, H), jnp.float32),
         jax.ShapeDtypeStruct((N + 8, H), jnp.float32)),
        S1[0], S1[1], h1p, dinv, b1r, g1r, be1r, W2)

    S2 = scat_k(h2p, src_m, dst_m, zerosNH)
    _, h3p = _tc_call(
        _tc_mid_body,
        (jax.ShapeDtypeStruct((N, H), jnp.float32),
         jax.ShapeDtypeStruct((N + 8, H), jnp.float32)),
        S2[0], S2[1], h2p, dinv, b2r, g2r, be2r, W3)

    S3 = scat_k(h3p, src_m, dst_m, zerosNH)
    out = _tc_call(
        _tc_final_body,
        jax.ShapeDtypeStruct((N, 1), jnp.float32),
        S3[0], S3[1], h3p, dinv, b3r, g3r, be3r, h1, Wfc, bfcr)

    return out


# speedup vs baseline: 1.0155x; 1.0155x over previous
import functools

import jax
import jax.numpy as jnp
from jax import lax
from jax.experimental import pallas as pl
from jax.experimental.pallas import tpu as pltpu
from jax.experimental.pallas import tpu_sc as plsc

_EPS = 1e-5
_NC, _NS = 2, 16
_NW = _NC * _NS
_CW = 128


def _sc_mesh():
    return plsc.VectorSubcoreMesh(core_axis_name="c", subcore_axis_name="s")


def _row_split(N):
    rows_a = (N // _NS) & ~7
    tail = N - _NS * rows_a
    return rows_a, tail


def _make_deg1_kernel(N, n_pw):
    rows_a, tail = _row_split(N)
    Np = N + 8
    n_half = n_pw // 2

    @functools.partial(
        pl.kernel,
        out_type=jax.ShapeDtypeStruct((_NC * N,), jnp.float32),
        mesh=_sc_mesh(),
        scratch_types=[
            pltpu.VMEM((n_half, _CW), jnp.int32),
            pltpu.VMEM((_CW,), jnp.float32),
            pltpu.VMEM((rows_a + tail + 8,), jnp.float32),
            pltpu.VMEM_SHARED((Np,), jnp.float32),
            pltpu.SemaphoreType.DMA,
            pltpu.SemaphoreType.DMA,
        ],
    )
    def deg_kernel(ones_hbm, dst_hbm, zeros_hbm, out_hbm,
                   dst_v, ones_v, stage_v, acc_sh, ss0, ss1):
        c = lax.axis_index("c")
        s = lax.axis_index("s")
        w = s * _NC + c
        t0 = s * rows_a
        pltpu.sync_copy(zeros_hbm.at[pl.ds(0, rows_a + tail + 8)], stage_v)
        pltpu.sync_copy(stage_v.at[pl.ds(0, rows_a)],
                        acc_sh.at[pl.ds(t0, rows_a)])

        @pl.when(s == _NS - 1)
        def _():
            pltpu.sync_copy(stage_v.at[pl.ds(rows_a, tail + 8)],
                            acc_sh.at[pl.ds(_NS * rows_a, tail + 8)])

        pltpu.sync_copy(ones_hbm, ones_v)
        plsc.subcore_barrier()
        sems = (ss0, ss1)

        def issue(i, b):
            pltpu.async_copy(ones_v, acc_sh.at[dst_v.at[i]], sems[b], add=True)

        def wait(i, b):
            pltpu.make_async_copy(ones_v, acc_sh.at[dst_v.at[i]],
                                  sems[b]).wait()

        def run_half(half):
            row_base = w * n_pw + half * n_half
            pltpu.sync_copy(dst_hbm.at[pl.ds(row_base, n_half)], dst_v)
            issue(0, 0)
            issue(1, 1)

            def body(p, carry):
                for q in range(2):
                    i = 2 * p + q + 2
                    wait(i - 2, q)
                    issue(i, q)
                return carry

            lax.fori_loop(0, (n_half - 2) // 2, body, 0)
            for i in range(2 + 2 * ((n_half - 2) // 2), n_half):
                wait(i - 2, i % 2)
                issue(i, i % 2)
            for i in range(n_half - 2, n_half):
                wait(i, i % 2)

        run_half(0)
        run_half(1)
        plsc.subcore_barrier()
        pltpu.sync_copy(acc_sh.at[pl.ds(t0, rows_a)],
                        stage_v.at[pl.ds(0, rows_a)])
        pltpu.sync_copy(stage_v.at[pl.ds(0, rows_a)],
                        out_hbm.at[pl.ds(c * N + t0, rows_a)])

        @pl.when(s == _NS - 1)
        def _():
            pltpu.sync_copy(acc_sh.at[pl.ds(_NS * rows_a, tail)],
                            stage_v.at[pl.ds(0, tail)])
            pltpu.sync_copy(stage_v.at[pl.ds(0, tail)],
                            out_hbm.at[pl.ds(c * N + _NS * rows_a, tail)])

    return deg_kernel


def _split_pass(n):
    p0 = ((n // 2) + 7) & ~7
    return p0, n - p0


def _make_scatter_kernel(N, n_pw, H, n_c0=None):
    rows_a, tail = _row_split(N)
    Np = N + 8
    if n_c0 is None:
        n_c0 = n_pw
    n_c1 = 2 * n_pw - n_c0
    passes0 = _split_pass(n_c0)
    passes1 = _split_pass(n_c1)
    max_pass = max(passes0 + passes1)

    scratch = [
        pltpu.VMEM((max_pass, _CW), jnp.int32),
        pltpu.VMEM((max_pass, _CW), jnp.int32),
        pltpu.VMEM((_CW, H), jnp.float32),
        pltpu.VMEM((_CW, H), jnp.float32),
        pltpu.VMEM_SHARED((Np, H), jnp.float32),
        pltpu.SemaphoreType.DMA,
        pltpu.SemaphoreType.DMA,
        pltpu.SemaphoreType.DMA,
        pltpu.SemaphoreType.DMA,
    ]

    @functools.partial(
        pl.kernel,
        out_type=jax.ShapeDtypeStruct((_NC, N, H), jnp.float32),
        mesh=_sc_mesh(),
        scratch_types=scratch,
    )
    def scatter_kernel(hp_hbm, src_hbm, dst_hbm, zeros_hbm, out_hbm,
                       src_v, dst_v, r0b, r1b, acc_sh, sg0, sg1, ss0, ss1):
        c = lax.axis_index("c")
        s = lax.axis_index("s")
        t0 = s * rows_a
        pltpu.sync_copy(zeros_hbm.at[pl.ds(t0, rows_a)],
                        acc_sh.at[pl.ds(t0, rows_a)])

        @pl.when(s == _NS - 1)
        def _():
            pltpu.sync_copy(zeros_hbm.at[pl.ds(_NS * rows_a, tail)],
                            acc_sh.at[pl.ds(_NS * rows_a, tail)])

        plsc.subcore_barrier()

        gbufs = ((r0b, sg0, ss0), (r1b, sg1, ss1))

        def start_gather(i, b):
            rows_v, sg, _ = gbufs[b]
            pltpu.async_copy(hp_hbm.at[src_v.at[i]], rows_v, sg)

        def wait_gather(i, b):
            rows_v, sg, _ = gbufs[b]
            pltpu.make_async_copy(hp_hbm.at[src_v.at[i]], rows_v, sg).wait()

        def issue_scatter(i, b):
            rows_v, _, ss = gbufs[b]
            pltpu.async_copy(rows_v, acc_sh.at[dst_v.at[i]], ss, add=True)

        def wait_scatter(i, b):
            rows_v, _, ss = gbufs[b]
            pltpu.make_async_copy(rows_v, acc_sh.at[dst_v.at[i]], ss).wait()

        def run_pass(row_base, n_pass):
            if n_pass == 0:
                return
            pltpu.sync_copy(dst_hbm.at[pl.ds(row_base, n_pass)],
                            dst_v.at[pl.ds(0, n_pass)])
            pltpu.sync_copy(src_hbm.at[pl.ds(row_base, n_pass)],
                            src_v.at[pl.ds(0, n_pass)])

            start_gather(0, 0)
            wait_gather(0, 0)
            issue_scatter(0, 0)
            if n_pass == 1:
                wait_scatter(0, 0)
                return
            start_gather(1, 1)

            def mid(i, b):
                bp = b ^ 1
                wait_gather(i, b)
                issue_scatter(i, b)
                wait_scatter(i - 1, bp)

                @pl.when(i + 1 < n_pass)
                def _():
                    start_gather(i + 1, bp)

            def body(p, carry):
                mid(2 * p + 1, 1)
                mid(2 * p + 2, 0)
                return carry

            lax.fori_loop(0, (n_pass - 1) // 2, body, 0)
            for i in range(1 + 2 * ((n_pass - 1) // 2), n_pass):
                mid(i, i % 2)
            wait_scatter(n_pass - 1, (n_pass - 1) % 2)

        def run_core(n_core, passes):
            base = s * (n_c0 + n_c1) + jnp.where(c == 0, 0, n_c0)
            run_pass(base, passes[0])
            run_pass(base + passes[0], passes[1])

        if passes0 == passes1:
            run_core(n_c0, passes0)
        else:
            @pl.when(c == 0)
            def _():
                run_core(n_c0, passes0)

            @pl.when(c == 1)
            def _():
                run_core(n_c1, passes1)

        plsc.subcore_barrier()
        pltpu.sync_copy(acc_sh.at[pl.ds(t0, rows_a)],
                        out_hbm.at[c, pl.ds(t0, rows_a)])

        @pl.when(s == _NS - 1)
        def _():
            pltpu.sync_copy(acc_sh.at[pl.ds(_NS * rows_a, tail)],
                            out_hbm.at[c, pl.ds(_NS * rows_a, tail)])

    return scatter_kernel


def _tc0_body(x_ref, w_ref, d0_ref, d1_ref, hp_ref, dinv_ref):
    N = x_ref.shape[0]
    deg = d0_ref[:, 0:1] + d1_ref[:, 0:1] + 1.0
    dinv = lax.rsqrt(deg)
    dinv_ref[...] = dinv
    pre = jnp.dot(x_ref[...], w_ref[...], preferred_element_type=jnp.float32)
    hp_ref[pl.ds(0, N), :] = pre * dinv
    hp_ref[pl.ds(N, 8), :] = jnp.zeros((8, hp_ref.shape[1]), jnp.float32)


def _tc_mid_body(s0_ref, s1_ref, hp_ref, dinv_ref, b_ref, g_ref, be_ref,
                 w_ref, hn_ref, hnp_ref):
    N = s0_ref.shape[0]
    dinv = dinv_ref[...]
    z = dinv * (s0_ref[...] + s1_ref[...] + hp_ref[pl.ds(0, N), :]) + b_ref[...]
    mu = jnp.mean(z, axis=0, keepdims=True)
    var = jnp.mean((z - mu) ** 2, axis=0, keepdims=True)
    hn = jnp.maximum((z - mu) * lax.rsqrt(var + _EPS) * g_ref[...] + be_ref[...],
                     0.0)
    hn_ref[...] = hn
    hnp_ref[pl.ds(0, N), :] = jnp.dot(hn, w_ref[...],
                                      preferred_element_type=jnp.float32) * dinv
    hnp_ref[pl.ds(N, 8), :] = jnp.zeros((8, hnp_ref.shape[1]), jnp.float32)


def _tc_final_body(s0_ref, s1_ref, hp_ref, dinv_ref, b_ref, g_ref, be_ref,
                   res_ref, wfc_ref, bfc_ref, out_ref):
    N = s0_ref.shape[0]
    dinv = dinv_ref[...]
    z = dinv * (s0_ref[...] + s1_ref[...] + hp_ref[pl.ds(0, N), :]) + b_ref[...]
    mu = jnp.mean(z, axis=0, keepdims=True)
    var = jnp.mean((z - mu) ** 2, axis=0, keepdims=True)
    h = jnp.maximum((z - mu) * lax.rsqrt(var + _EPS) * g_ref[...] + be_ref[...],
                    0.0)
    h = h + res_ref[...]
    logits = jnp.dot(h, wfc_ref[...], preferred_element_type=jnp.float32)
    out_ref[...] = jax.nn.sigmoid(logits + bfc_ref[...]) * 10.0


def _tc_call(body, out_shapes, *args):
    return pl.pallas_call(body, out_shape=out_shapes)(*args)


def kernel(x, edge_index, W1, b1, g1, be1, W2, b2, g2, be2, W3, b3, g3, be3,
           Wfc, bfc):
    N, D = x.shape
    H = W1.shape[1]
    E = edge_index.shape[1]

    grain = _NW * _CW * 8
    E_pad = -(-E // grain) * grain
    pad = E_pad - E
    pad_src = N + (jnp.arange(pad, dtype=jnp.int32) % 8)
    pad_dst = (jnp.arange(pad, dtype=jnp.int32) * 997) % N
    src_m = jnp.concatenate([edge_index[0], pad_src]).reshape(E_pad // _CW, _CW)
    dst_m = jnp.concatenate([edge_index[1], pad_dst]).reshape(E_pad // _CW, _CW)
    dst_deg = jnp.concatenate(
        [edge_index[1], pad_src]).reshape(E_pad // _CW, _CW)
    n_pw = E_pad // _CW // _NW

    zerosNH = jnp.zeros((N, H), jnp.float32)
    zeros1 = jnp.zeros((N + 8,), jnp.float32)
    ones1 = jnp.ones((_CW,), jnp.float32)

    b1r, g1r, be1r = b1.reshape(1, H), g1.reshape(1, H), be1.reshape(1, H)
    b2r, g2r, be2r = b2.reshape(1, H), g2.reshape(1, H), be2.reshape(1, H)
    b3r, g3r, be3r = b3.reshape(1, H), g3.reshape(1, H), be3.reshape(1, H)
    bfcr = bfc.reshape(1, 1)

    deg_k = _make_deg1_kernel(N, n_pw)
    scat_k = _make_scatter_kernel(N, n_pw, H, n_c0=72)

    degp = deg_k(ones1, dst_deg, zeros1).reshape(_NC, N, 1)

    h1p, dinv = _tc_call(
        _tc0_body,
        (jax.ShapeDtypeStruct((N + 8, H), jnp.float32),
         jax.ShapeDtypeStruct((N, 1), jnp.float32)),
        x, W1, degp[0], degp[1])

    S1 = scat_k(h1p, src_m, dst_m, zerosNH)
    h1, h2p = _tc_call(
        _tc_mid_body,
        (jax.ShapeDtypeStruct((N, H), jnp.float32),
         jax.ShapeDtypeStruct((N + 8, H), jnp.float32)),
        S1[0], S1[1], h1p, dinv, b1r, g1r, be1r, W2)

    S2 = scat_k(h2p, src_m, dst_m, zerosNH)
    _, h3p = _tc_call(
        _tc_mid_body,
        (jax.ShapeDtypeStruct((N, H), jnp.float32),
         jax.ShapeDtypeStruct((N + 8, H), jnp.float32)),
        S2[0], S2[1], h2p, dinv, b2r, g2r, be2r, W3)

    S3 = scat_k(h3p, src_m, dst_m, zerosNH)
    out = _tc_call(
        _tc_final_body,
        jax.ShapeDtypeStruct((N, 1), jnp.float32),
        S3[0], S3[1], h3p, dinv, b3r, g3r, be3r, h1, Wfc, bfcr)

    return out


# speedup vs baseline: 1.1060x; 1.0891x over previous
import functools

import jax
import jax.numpy as jnp
from jax import lax
from jax.experimental import pallas as pl
from jax.experimental.pallas import tpu as pltpu
from jax.experimental.pallas import tpu_sc as plsc

_EPS = 1e-5
_NC, _NS = 2, 16
_NW = _NC * _NS
_CW = 128


def _sc_mesh():
    return plsc.VectorSubcoreMesh(core_axis_name="c", subcore_axis_name="s")


def _row_split(N):
    rows_a = (N // _NS) & ~7
    tail = N - _NS * rows_a
    return rows_a, tail


def _make_deg1_kernel(N, n_pw):
    rows_a, tail = _row_split(N)
    Np = N + 8
    n_half = n_pw // 2

    @functools.partial(
        pl.kernel,
        out_type=jax.ShapeDtypeStruct((_NC * N,), jnp.float32),
        mesh=_sc_mesh(),
        scratch_types=[
            pltpu.VMEM((n_half, _CW), jnp.int32),
            pltpu.VMEM((_CW,), jnp.float32),
            pltpu.VMEM((rows_a + tail + 8,), jnp.float32),
            pltpu.VMEM_SHARED((Np,), jnp.float32),
            pltpu.SemaphoreType.DMA,
            pltpu.SemaphoreType.DMA,
        ],
    )
    def deg_kernel(ones_hbm, dst_hbm, zeros_hbm, out_hbm,
                   dst_v, ones_v, stage_v, acc_sh, ss0, ss1):
        c = lax.axis_index("c")
        s = lax.axis_index("s")
        w = s * _NC + c
        t0 = s * rows_a
        pltpu.sync_copy(zeros_hbm.at[pl.ds(0, rows_a + tail + 8)], stage_v)
        pltpu.sync_copy(stage_v.at[pl.ds(0, rows_a)],
                        acc_sh.at[pl.ds(t0, rows_a)])

        @pl.when(s == _NS - 1)
        def _():
            pltpu.sync_copy(stage_v.at[pl.ds(rows_a, tail + 8)],
                            acc_sh.at[pl.ds(_NS * rows_a, tail + 8)])

        pltpu.sync_copy(ones_hbm, ones_v)
        plsc.subcore_barrier()
        sems = (ss0, ss1)

        def issue(i, b):
            pltpu.async_copy(ones_v, acc_sh.at[dst_v.at[i]], sems[b], add=True)

        def wait(i, b):
            pltpu.make_async_copy(ones_v, acc_sh.at[dst_v.at[i]],
                                  sems[b]).wait()

        def run_half(half):
            row_base = w * n_pw + half * n_half
            pltpu.sync_copy(dst_hbm.at[pl.ds(row_base, n_half)], dst_v)
            issue(0, 0)
            issue(1, 1)

            def body(p, carry):
                for q in range(2):
                    i = 2 * p + q + 2
                    wait(i - 2, q)
                    issue(i, q)
                return carry

            lax.fori_loop(0, (n_half - 2) // 2, body, 0)
            for i in range(2 + 2 * ((n_half - 2) // 2), n_half):
                wait(i - 2, i % 2)
                issue(i, i % 2)
            for i in range(n_half - 2, n_half):
                wait(i, i % 2)

        run_half(0)
        run_half(1)
        plsc.subcore_barrier()
        pltpu.sync_copy(acc_sh.at[pl.ds(t0, rows_a)],
                        stage_v.at[pl.ds(0, rows_a)])
        pltpu.sync_copy(stage_v.at[pl.ds(0, rows_a)],
                        out_hbm.at[pl.ds(c * N + t0, rows_a)])

        @pl.when(s == _NS - 1)
        def _():
            pltpu.sync_copy(acc_sh.at[pl.ds(_NS * rows_a, tail)],
                            stage_v.at[pl.ds(0, tail)])
            pltpu.sync_copy(stage_v.at[pl.ds(0, tail)],
                            out_hbm.at[pl.ds(c * N + _NS * rows_a, tail)])

    return deg_kernel


def _split_pass(n):
    p0 = ((n // 2) + 7) & ~7
    return p0, n - p0


def _make_scatter_kernel(N, n_pw, H, n_c0=None):
    rows_a, tail = _row_split(N)
    Np = N + 8
    if n_c0 is None:
        n_c0 = n_pw
    n_c1 = 2 * n_pw - n_c0
    passes0 = _split_pass(n_c0)
    passes1 = _split_pass(n_c1)
    max_pass = max(passes0 + passes1)

    scratch = [
        pltpu.VMEM((max_pass, _CW), jnp.int32),
        pltpu.VMEM((max_pass, _CW), jnp.int32),
        pltpu.VMEM((_CW, H), jnp.float32),
        pltpu.VMEM((_CW, H), jnp.float32),
        pltpu.VMEM_SHARED((Np, H), jnp.float32),
        pltpu.SemaphoreType.DMA,
        pltpu.SemaphoreType.DMA,
        pltpu.SemaphoreType.DMA,
        pltpu.SemaphoreType.DMA,
    ]

    @functools.partial(
        pl.kernel,
        out_type=jax.ShapeDtypeStruct((_NC, N, H), jnp.float32),
        mesh=_sc_mesh(),
        scratch_types=scratch,
    )
    def scatter_kernel(hp_hbm, src_hbm, dst_hbm, zeros_hbm, out_hbm,
                       src_v, dst_v, r0b, r1b, acc_sh, sg0, sg1, ss0, ss1):
        c = lax.axis_index("c")
        s = lax.axis_index("s")
        t0 = s * rows_a
        pltpu.sync_copy(zeros_hbm.at[pl.ds(t0, rows_a)],
                        acc_sh.at[pl.ds(t0, rows_a)])

        @pl.when(s == _NS - 1)
        def _():
            pltpu.sync_copy(zeros_hbm.at[pl.ds(_NS * rows_a, tail)],
                            acc_sh.at[pl.ds(_NS * rows_a, tail)])

        plsc.subcore_barrier()

        gbufs = ((r0b, sg0, ss0), (r1b, sg1, ss1))

        def start_gather(i, b):
            rows_v, sg, _ = gbufs[b]
            pltpu.async_copy(hp_hbm.at[src_v.at[i]], rows_v, sg)

        def wait_gather(i, b):
            rows_v, sg, _ = gbufs[b]
            pltpu.make_async_copy(hp_hbm.at[src_v.at[i]], rows_v, sg).wait()

        def issue_scatter(i, b):
            rows_v, _, ss = gbufs[b]
            pltpu.async_copy(rows_v, acc_sh.at[dst_v.at[i]], ss, add=True)

        def wait_scatter(i, b):
            rows_v, _, ss = gbufs[b]
            pltpu.make_async_copy(rows_v, acc_sh.at[dst_v.at[i]], ss).wait()

        def run_pass(row_base, n_pass):
            if n_pass == 0:
                return
            pltpu.sync_copy(dst_hbm.at[pl.ds(row_base, n_pass)],
                            dst_v.at[pl.ds(0, n_pass)])
            pltpu.sync_copy(src_hbm.at[pl.ds(row_base, n_pass)],
                            src_v.at[pl.ds(0, n_pass)])

            start_gather(0, 0)
            wait_gather(0, 0)
            issue_scatter(0, 0)
            if n_pass == 1:
                wait_scatter(0, 0)
                return
            start_gather(1, 1)

            def mid(i, b):
                bp = b ^ 1
                wait_gather(i, b)
                issue_scatter(i, b)
                wait_scatter(i - 1, bp)

                @pl.when(i + 1 < n_pass)
                def _():
                    start_gather(i + 1, bp)

            def body(p, carry):
                mid(2 * p + 1, 1)
                mid(2 * p + 2, 0)
                return carry

            lax.fori_loop(0, (n_pass - 1) // 2, body, 0)
            for i in range(1 + 2 * ((n_pass - 1) // 2), n_pass):
                mid(i, i % 2)
            wait_scatter(n_pass - 1, (n_pass - 1) % 2)

        def run_core(n_core, passes):
            base = s * (n_c0 + n_c1) + jnp.where(c == 0, 0, n_c0)
            run_pass(base, passes[0])
            run_pass(base + passes[0], passes[1])

        if passes0 == passes1:
            run_core(n_c0, passes0)
        else:
            @pl.when(c == 0)
            def _():
                run_core(n_c0, passes0)

            @pl.when(c == 1)
            def _():
                run_core(n_c1, passes1)

        plsc.subcore_barrier()
        pltpu.sync_copy(acc_sh.at[pl.ds(t0, rows_a)],
                        out_hbm.at[c, pl.ds(t0, rows_a)])

        @pl.when(s == _NS - 1)
        def _():
            pltpu.sync_copy(acc_sh.at[pl.ds(_NS * rows_a, tail)],
                            out_hbm.at[c, pl.ds(_NS * rows_a, tail)])

    return scatter_kernel


def _tc0_body(x_ref, w_ref, d0_ref, d1_ref, hp_ref, dinv_ref):
    N = x_ref.shape[0]
    deg = d0_ref[:, 0:1] + d1_ref[:, 0:1] + 1.0
    dinv = lax.rsqrt(deg)
    dinv_ref[...] = dinv
    pre = jnp.dot(x_ref[...], w_ref[...], preferred_element_type=jnp.float32)
    hp_ref[pl.ds(0, N), :] = pre * dinv
    hp_ref[pl.ds(N, 8), :] = jnp.zeros((8, hp_ref.shape[1]), jnp.float32)


def _tc_mid_body(s0_ref, s1_ref, hp_ref, dinv_ref, b_ref, g_ref, be_ref,
                 w_ref, hn_ref, hnp_ref):
    N = s0_ref.shape[0]
    dinv = dinv_ref[...]
    z = dinv * (s0_ref[...] + s1_ref[...] + hp_ref[pl.ds(0, N), :]) + b_ref[...]
    mu = jnp.mean(z, axis=0, keepdims=True)
    var = jnp.mean((z - mu) ** 2, axis=0, keepdims=True)
    hn = jnp.maximum((z - mu) * lax.rsqrt(var + _EPS) * g_ref[...] + be_ref[...],
                     0.0)
    hn_ref[...] = hn
    hnp_ref[pl.ds(0, N), :] = jnp.dot(hn, w_ref[...],
                                      preferred_element_type=jnp.float32) * dinv
    hnp_ref[pl.ds(N, 8), :] = jnp.zeros((8, hnp_ref.shape[1]), jnp.float32)


def _tc_final_body(s0_ref, s1_ref, hp_ref, dinv_ref, b_ref, g_ref, be_ref,
                   res_ref, wfc_ref, bfc_ref, out_ref):
    N = s0_ref.shape[0]
    dinv = dinv_ref[...]
    z = dinv * (s0_ref[...] + s1_ref[...] + hp_ref[pl.ds(0, N), :]) + b_ref[...]
    mu = jnp.mean(z, axis=0, keepdims=True)
    var = jnp.mean((z - mu) ** 2, axis=0, keepdims=True)
    h = jnp.maximum((z - mu) * lax.rsqrt(var + _EPS) * g_ref[...] + be_ref[...],
                    0.0)
    h = h + res_ref[...]
    logits = jnp.dot(h, wfc_ref[...], preferred_element_type=jnp.float32)
    out_ref[...] = jax.nn.sigmoid(logits + bfc_ref[...]) * 10.0


def _tc_call(body, out_shapes, *args):
    return pl.pallas_call(body, out_shape=out_shapes)(*args)


def kernel(x, edge_index, W1, b1, g1, be1, W2, b2, g2, be2, W3, b3, g3, be3,
           Wfc, bfc):
    N, D = x.shape
    H = W1.shape[1]
    E = edge_index.shape[1]

    grain = _NW * _CW * 8
    E_pad = -(-E // grain) * grain
    pad = E_pad - E
    pad_src = N + (jnp.arange(pad, dtype=jnp.int32) % 8)
    pad_dst = (jnp.arange(pad, dtype=jnp.int32) * 997) % N
    src_m = jnp.concatenate([edge_index[0], pad_src]).reshape(E_pad // _CW, _CW)
    dst_m = jnp.concatenate([edge_index[1], pad_dst]).reshape(E_pad // _CW, _CW)
    dst_deg = jnp.concatenate(
        [edge_index[1], pad_src]).reshape(E_pad // _CW, _CW)
    n_pw = E_pad // _CW // _NW

    zerosNH = jnp.zeros((N, H), jnp.float32)
    zeros1 = jnp.zeros((N + 8,), jnp.float32)
    ones1 = jnp.ones((_CW,), jnp.float32)

    b1r, g1r, be1r = b1.reshape(1, H), g1.reshape(1, H), be1.reshape(1, H)
    b2r, g2r, be2r = b2.reshape(1, H), g2.reshape(1, H), be2.reshape(1, H)
    b3r, g3r, be3r = b3.reshape(1, H), g3.reshape(1, H), be3.reshape(1, H)
    bfcr = bfc.reshape(1, 1)

    deg_k = _make_deg1_kernel(N, n_pw)
    scat_k = _make_scatter_kernel(N, n_pw, H, n_c0=88)

    degp = deg_k(ones1, dst_deg, zeros1).reshape(_NC, N, 1)

    h1p, dinv = _tc_call(
        _tc0_body,
        (jax.ShapeDtypeStruct((N + 8, H), jnp.float32),
         jax.ShapeDtypeStruct((N, 1), jnp.float32)),
        x, W1, degp[0], degp[1])

    S1 = scat_k(h1p, src_m, dst_m, zerosNH)
    h1, h2p = _tc_call(
        _tc_mid_body,
        (jax.ShapeDtypeStruct((N, H), jnp.float32),
         jax.ShapeDtypeStruct((N + 8, H), jnp.float32)),
        S1[0], S1[1], h1p, dinv, b1r, g1r, be1r, W2)

    S2 = scat_k(h2p, src_m, dst_m, zerosNH)
    _, h3p = _tc_call(
        _tc_mid_body,
        (jax.ShapeDtypeStruct((N, H), jnp.float32),
         jax.ShapeDtypeStruct((N + 8, H), jnp.float32)),
        S2[0], S2[1], h2p, dinv, b2r, g2r, be2r, W3)

    S3 = scat_k(h3p, src_m, dst_m, zerosNH)
    out = _tc_call(
        _tc_final_body,
        jax.ShapeDtypeStruct((N, 1), jnp.float32),
        S3[0], S3[1], h3p, dinv, b3r, g3r, be3r, h1, Wfc, bfcr)

    return out
